# baseline (speedup 1.0000x reference)
import functools

import jax
import jax.numpy as jnp
from jax import lax
from jax.experimental import pallas as pl
from jax.experimental.pallas import tpu as pltpu
from jax.experimental.pallas import tpu_sc as plsc

_N = 99999
_K = _N // 3
_NC = 2
_NS = 16
_NW = _NC * _NS
_CHUNK = 8192
_NV = _CHUNK // 16
_XPW = 50304
_ACC_P = 33408
_INV3 = 0xAAAAAAAB
_LIM3 = 0x55555555


def _make_sc_kernel(nch):
    mesh = plsc.VectorSubcoreMesh(core_axis_name="c", subcore_axis_name="s",
                                  num_cores=_NC, num_subcores=_NS)

    @functools.partial(
        pl.kernel,
        out_type=jax.ShapeDtypeStruct((_NW * _ACC_P,), jnp.float32),
        mesh=mesh,
        compiler_params=pltpu.CompilerParams(
            needs_layout_passes=False, use_tc_tiling_on_sc=False),
        scratch_types=[
            pltpu.VMEM((_XPW,), jnp.int32),
            pltpu.VMEM((2, _CHUNK), jnp.int32),
            pltpu.VMEM((2, _CHUNK), jnp.int32),
            pltpu.VMEM((_ACC_P,), jnp.float32),
            pltpu.SemaphoreType.DMA,
            pltpu.SemaphoreType.DMA,
            pltpu.SemaphoreType.DMA,
            pltpu.SemaphoreType.DMA,
        ],
    )
    def sc_kernel(xp_hbm, srcr_hbm, dstr_hbm, zeros_hbm, q_hbm,
                  xp_v, src_v, dst_v, acc_v,
                  sem_s0, sem_s1, sem_d0, sem_d1):
        cid = lax.axis_index("c")
        sid = lax.axis_index("s")
        wid = sid * _NC + cid
        sem_s = (sem_s0, sem_s1)
        sem_d = (sem_d0, sem_d1)

        pltpu.sync_copy(xp_hbm, xp_v)
        pltpu.sync_copy(zeros_hbm, acc_v)

        row0 = wid * nch
        inv3 = jnp.full((16,), _INV3, dtype=jnp.uint32)
        lim3 = jnp.full((16,), _LIM3, dtype=jnp.uint32)
        himask = jnp.full((16,), 0xFFFF0000, dtype=jnp.uint32)
        one = jnp.full((16,), 1, dtype=jnp.int32)

        _PC = _CHUNK // 4

        def start(g, b):
            for pc in range(4):
                pltpu.async_copy(
                    srcr_hbm.at[row0 + g, pl.ds(pc * _PC, _PC)],
                    src_v.at[b, pl.ds(pc * _PC, _PC)], sem_s[b])
                pltpu.async_copy(
                    dstr_hbm.at[row0 + g, pl.ds(pc * _PC, _PC)],
                    dst_v.at[b, pl.ds(pc * _PC, _PC)], sem_d[b])

        def wait(g, b):
            for pc in range(4):
                pltpu.make_async_copy(
                    srcr_hbm.at[row0 + g, pl.ds(pc * _PC, _PC)],
                    src_v.at[b, pl.ds(pc * _PC, _PC)], sem_s[b]).wait()
                pltpu.make_async_copy(
                    dstr_hbm.at[row0 + g, pl.ds(pc * _PC, _PC)],
                    dst_v.at[b, pl.ds(pc * _PC, _PC)], sem_d[b]).wait()

        def process(b):
            @plsc.parallel_loop(0, _NV, 1, unroll=8)
            def _body(u):
                off = u * 16
                d = dst_v[b, pl.ds(off, 16)]
                s = src_v[b, pl.ds(off, 16)]
                t = plsc.bitcast(d, jnp.uint32) * inv3
                m = t <= lim3
                pair = plsc.load_gather(xp_v, [lax.shift_right_logical(s, 1)])
                pu = plsc.bitcast(pair, jnp.uint32)
                sel = jnp.bitwise_and(s, one)
                bits = jnp.where(sel == one, pu & himask,
                                 lax.shift_left(pu, jnp.uint32(16)))
                v = plsc.bitcast(bits, jnp.float32)
                q = plsc.bitcast(t, jnp.int32)
                plsc.addupdate_scatter(acc_v, [q], v, mask=m)

        start(0, 0)
        start(1, 1)

        def outer(i, carry):
            for b in range(2):
                g = 2 * i + b
                wait(g, b)
                process(b)

                @pl.when(g + 2 < nch)
                def _():
                    start(g + 2, b)
            return carry

        lax.fori_loop(0, nch // 2, outer, 0)

        pltpu.sync_copy(acc_v, q_hbm.at[pl.ds(wid * _ACC_P, _ACC_P)])

    return sc_kernel


def _combine_body(p_ref, q_ref, z_ref, o_ref):
    agg = jnp.sum(q_ref[...], axis=0)
    h = p_ref[0] * agg + p_ref[1] * z_ref[...]
    a0 = jnp.maximum(p_ref[2] * h + p_ref[3], 0.0)
    a1 = jnp.maximum(p_ref[4] * h + p_ref[5], 0.0)
    o_ref[...] = p_ref[6] * a0 + p_ref[7] * a1 + p_ref[8]


def kernel(x, edge_index, W_l, W_r, W1, b1, W2, b2):
    x1 = x.reshape(-1)
    src = edge_index[0]
    dst = edge_index[1]
    e = src.shape[0]

    per_worker = _NW * _CHUNK
    nch = -(-e // per_worker)
    nch = -(-nch // 2) * 2
    ep = nch * per_worker
    pad = ep - e

    src_p = jnp.concatenate([src, jnp.zeros((pad,), jnp.int32)])
    dst_p = jnp.concatenate([dst, jnp.ones((pad,), jnp.int32)])
    srcr = src_p.reshape(ep // _CHUNK, _CHUNK)
    dstr = dst_p.reshape(ep // _CHUNK, _CHUNK)

    xb = x1.astype(jnp.bfloat16)
    xu = lax.bitcast_convert_type(xb, jnp.uint16).astype(jnp.uint32)
    xu = jnp.concatenate([xu, jnp.zeros((2 * _XPW - _N,), jnp.uint32)])
    xp = xu[0::2] | (xu[1::2] << 16)
    xp = lax.bitcast_convert_type(xp, jnp.int32)
    zeros = jnp.zeros((_ACC_P,), jnp.float32)

    q = _make_sc_kernel(nch)(xp, srcr, dstr, zeros)

    xz = jnp.concatenate([x1[::3], jnp.zeros((_ACC_P - _K,), jnp.float32)])

    params = jnp.stack([
        W_l[0, 0], W_r[0, 0],
        W1[0, 0], b1[0], W1[1, 0], b1[1],
        W2[0, 0], W2[0, 1], b2[0],
    ])
    out2d = pl.pallas_call(
        _combine_body,
        out_shape=jax.ShapeDtypeStruct((_ACC_P // 128, 128), jnp.float32),
        in_specs=[
            pl.BlockSpec(memory_space=pltpu.SMEM),
            pl.BlockSpec(memory_space=pltpu.VMEM),
            pl.BlockSpec(memory_space=pltpu.VMEM),
        ],
    )(params, q.reshape(_NW, _ACC_P // 128, 128),
      xz.reshape(_ACC_P // 128, 128))
    return out2d.reshape(-1)[:_K]


# speedup vs baseline: 1.0141x; 1.0141x over previous
import functools

import jax
import jax.numpy as jnp
from jax import lax
from jax.experimental import pallas as pl
from jax.experimental.pallas import tpu as pltpu
from jax.experimental.pallas import tpu_sc as plsc

_N = 99999
_K = _N // 3
_NC = 2
_NS = 16
_NW = _NC * _NS
_CHUNK = 8192
_NV = _CHUNK // 16
_XPW = 50304
_ACC_P = 33408
_INV3 = 0xAAAAAAAB
_LIM3 = 0x55555555


def _make_sc_kernel(nch):
    mesh = plsc.VectorSubcoreMesh(core_axis_name="c", subcore_axis_name="s",
                                  num_cores=_NC, num_subcores=_NS)

    @functools.partial(
        pl.kernel,
        out_type=jax.ShapeDtypeStruct((_NW * _ACC_P,), jnp.float32),
        mesh=mesh,
        compiler_params=pltpu.CompilerParams(
            needs_layout_passes=False, use_tc_tiling_on_sc=False),
        scratch_types=[
            pltpu.VMEM((_XPW,), jnp.int32),
            pltpu.VMEM((2, _CHUNK), jnp.int32),
            pltpu.VMEM((2, _CHUNK), jnp.int32),
            pltpu.VMEM((_ACC_P,), jnp.float32),
            pltpu.SemaphoreType.DMA,
            pltpu.SemaphoreType.DMA,
            pltpu.SemaphoreType.DMA,
            pltpu.SemaphoreType.DMA,
        ],
    )
    def sc_kernel(xp_hbm, srcr_hbm, dstr_hbm, zeros_hbm, q_hbm,
                  xp_v, src_v, dst_v, acc_v,
                  sem_s0, sem_s1, sem_d0, sem_d1):
        cid = lax.axis_index("c")
        sid = lax.axis_index("s")
        wid = sid * _NC + cid
        sem_s = (sem_s0, sem_s1)
        sem_d = (sem_d0, sem_d1)

        row0 = wid * nch
        inv3 = jnp.full((16,), _INV3, dtype=jnp.uint32)
        lim3 = jnp.full((16,), _LIM3, dtype=jnp.uint32)
        himask = jnp.full((16,), 0xFFFF0000, dtype=jnp.uint32)
        one = jnp.full((16,), 1, dtype=jnp.int32)

        def start(g, b):
            pltpu.async_copy(srcr_hbm.at[row0 + g], src_v.at[b], sem_s[b])
            pltpu.async_copy(dstr_hbm.at[row0 + g], dst_v.at[b], sem_d[b])

        def wait(g, b):
            pltpu.make_async_copy(srcr_hbm.at[row0 + g], src_v.at[b],
                                  sem_s[b]).wait()
            pltpu.make_async_copy(dstr_hbm.at[row0 + g], dst_v.at[b],
                                  sem_d[b]).wait()

        def process(b):
            @plsc.parallel_loop(0, _NV, 1, unroll=8)
            def _body(u):
                off = u * 16
                d = dst_v[b, pl.ds(off, 16)]
                s = src_v[b, pl.ds(off, 16)]
                t = plsc.bitcast(d, jnp.uint32) * inv3
                m = t <= lim3
                pair = plsc.load_gather(xp_v, [lax.shift_right_logical(s, 1)])
                pu = plsc.bitcast(pair, jnp.uint32)
                sel = jnp.bitwise_and(s, one)
                bits = jnp.where(sel == one, pu & himask,
                                 lax.shift_left(pu, jnp.uint32(16)))
                v = plsc.bitcast(bits, jnp.float32)
                q = plsc.bitcast(t, jnp.int32)
                plsc.addupdate_scatter(acc_v, [q], v, mask=m)

        start(0, 0)
        start(1, 1)
        pltpu.sync_copy(xp_hbm, xp_v)
        pltpu.sync_copy(zeros_hbm, acc_v)

        def outer(i, carry):
            for b in range(2):
                g = 2 * i + b
                wait(g, b)
                process(b)

                @pl.when(g + 2 < nch)
                def _():
                    start(g + 2, b)
            return carry

        lax.fori_loop(0, nch // 2, outer, 0)

        pltpu.sync_copy(acc_v, q_hbm.at[pl.ds(wid * _ACC_P, _ACC_P)])

    return sc_kernel


def _combine_body(p_ref, q_ref, z_ref, o_ref):
    agg = jnp.sum(q_ref[...], axis=0)
    h = p_ref[0] * agg + p_ref[1] * z_ref[...]
    a0 = jnp.maximum(p_ref[2] * h + p_ref[3], 0.0)
    a1 = jnp.maximum(p_ref[4] * h + p_ref[5], 0.0)
    o_ref[...] = p_ref[6] * a0 + p_ref[7] * a1 + p_ref[8]


def kernel(x, edge_index, W_l, W_r, W1, b1, W2, b2):
    x1 = x.reshape(-1)
    src = edge_index[0]
    dst = edge_index[1]
    e = src.shape[0]

    per_worker = _NW * _CHUNK
    nch = -(-e // per_worker)
    nch = -(-nch // 2) * 2
    ep = nch * per_worker
    pad = ep - e

    src_p = jnp.concatenate([src, jnp.zeros((pad,), jnp.int32)])
    dst_p = jnp.concatenate([dst, jnp.ones((pad,), jnp.int32)])
    srcr = src_p.reshape(ep // _CHUNK, _CHUNK)
    dstr = dst_p.reshape(ep // _CHUNK, _CHUNK)

    xb = x1.astype(jnp.bfloat16)
    xu = lax.bitcast_convert_type(xb, jnp.uint16).astype(jnp.uint32)
    xu = jnp.concatenate([xu, jnp.zeros((2 * _XPW - _N,), jnp.uint32)])
    xp = xu[0::2] | (xu[1::2] << 16)
    xp = lax.bitcast_convert_type(xp, jnp.int32)
    zeros = jnp.zeros((_ACC_P,), jnp.float32)

    q = _make_sc_kernel(nch)(xp, srcr, dstr, zeros)

    xz = jnp.concatenate([x1[::3], jnp.zeros((_ACC_P - _K,), jnp.float32)])

    params = jnp.stack([
        W_l[0, 0], W_r[0, 0],
        W1[0, 0], b1[0], W1[1, 0], b1[1],
        W2[0, 0], W2[0, 1], b2[0],
    ])
    out2d = pl.pallas_call(
        _combine_body,
        out_shape=jax.ShapeDtypeStruct((_ACC_P // 128, 128), jnp.float32),
        in_specs=[
            pl.BlockSpec(memory_space=pltpu.SMEM),
            pl.BlockSpec(memory_space=pltpu.VMEM),
            pl.BlockSpec(memory_space=pltpu.VMEM),
        ],
    )(params, q.reshape(_NW, _ACC_P // 128, 128),
      xz.reshape(_ACC_P // 128, 128))
    return out2d.reshape(-1)[:_K]


# speedup vs baseline: 1.0270x; 1.0127x over previous
import functools

import jax
import jax.numpy as jnp
from jax import lax
from jax.experimental import pallas as pl
from jax.experimental.pallas import tpu as pltpu
from jax.experimental.pallas import tpu_sc as plsc

_N = 99999
_K = _N // 3
_NC = 2
_NS = 16
_NW = _NC * _NS
_CHUNK = 10240
_NV = _CHUNK // 16
_XPW = 50304
_ACC_P = 33408
_INV3 = 0xAAAAAAAB
_LIM3 = 0x55555555


def _make_sc_kernel(nch):
    mesh = plsc.VectorSubcoreMesh(core_axis_name="c", subcore_axis_name="s",
                                  num_cores=_NC, num_subcores=_NS)

    @functools.partial(
        pl.kernel,
        out_type=jax.ShapeDtypeStruct((_NW * _ACC_P,), jnp.float32),
        mesh=mesh,
        compiler_params=pltpu.CompilerParams(
            needs_layout_passes=False, use_tc_tiling_on_sc=False),
        scratch_types=[
            pltpu.VMEM((_XPW,), jnp.int32),
            pltpu.VMEM((2, _CHUNK), jnp.int32),
            pltpu.VMEM((2, _CHUNK), jnp.int32),
            pltpu.VMEM((_ACC_P,), jnp.float32),
            pltpu.SemaphoreType.DMA,
            pltpu.SemaphoreType.DMA,
            pltpu.SemaphoreType.DMA,
            pltpu.SemaphoreType.DMA,
        ],
    )
    def sc_kernel(xp_hbm, srcr_hbm, dstr_hbm, zeros_hbm, q_hbm,
                  xp_v, src_v, dst_v, acc_v,
                  sem_s0, sem_s1, sem_d0, sem_d1):
        cid = lax.axis_index("c")
        sid = lax.axis_index("s")
        wid = sid * _NC + cid
        sem_s = (sem_s0, sem_s1)
        sem_d = (sem_d0, sem_d1)

        row0 = wid * nch
        inv3 = jnp.full((16,), _INV3, dtype=jnp.uint32)
        lim3 = jnp.full((16,), _LIM3, dtype=jnp.uint32)
        himask = jnp.full((16,), 0xFFFF0000, dtype=jnp.uint32)
        one = jnp.full((16,), 1, dtype=jnp.int32)

        def start(g, b):
            pltpu.async_copy(srcr_hbm.at[row0 + g], src_v.at[b], sem_s[b])
            pltpu.async_copy(dstr_hbm.at[row0 + g], dst_v.at[b], sem_d[b])

        def wait(g, b):
            pltpu.make_async_copy(srcr_hbm.at[row0 + g], src_v.at[b],
                                  sem_s[b]).wait()
            pltpu.make_async_copy(dstr_hbm.at[row0 + g], dst_v.at[b],
                                  sem_d[b]).wait()

        def process(b):
            @plsc.parallel_loop(0, _NV, 1, unroll=8)
            def _body(u):
                off = u * 16
                d = dst_v[b, pl.ds(off, 16)]
                s = src_v[b, pl.ds(off, 16)]
                t = plsc.bitcast(d, jnp.uint32) * inv3
                m = t <= lim3
                pair = plsc.load_gather(xp_v, [lax.shift_right_logical(s, 1)])
                pu = plsc.bitcast(pair, jnp.uint32)
                sel = jnp.bitwise_and(s, one)
                bits = jnp.where(sel == one, pu & himask,
                                 lax.shift_left(pu, jnp.uint32(16)))
                v = plsc.bitcast(bits, jnp.float32)
                q = plsc.bitcast(t, jnp.int32)
                plsc.addupdate_scatter(acc_v, [q], v, mask=m)

        start(0, 0)
        start(1, 1)
        pltpu.sync_copy(xp_hbm, xp_v)
        pltpu.sync_copy(zeros_hbm, acc_v)

        def outer(i, carry):
            for b in range(2):
                g = 2 * i + b
                wait(g, b)
                process(b)

                @pl.when(g + 2 < nch)
                def _():
                    start(g + 2, b)
            return carry

        lax.fori_loop(0, nch // 2, outer, 0)

        pltpu.sync_copy(acc_v, q_hbm.at[pl.ds(wid * _ACC_P, _ACC_P)])

    return sc_kernel


def _combine_body(p_ref, q_ref, z_ref, o_ref):
    agg = jnp.sum(q_ref[...], axis=0)
    h = p_ref[0] * agg + p_ref[1] * z_ref[...]
    a0 = jnp.maximum(p_ref[2] * h + p_ref[3], 0.0)
    a1 = jnp.maximum(p_ref[4] * h + p_ref[5], 0.0)
    o_ref[...] = p_ref[6] * a0 + p_ref[7] * a1 + p_ref[8]


def kernel(x, edge_index, W_l, W_r, W1, b1, W2, b2):
    x1 = x.reshape(-1)
    src = edge_index[0]
    dst = edge_index[1]
    e = src.shape[0]

    per_worker = _NW * _CHUNK
    nch = -(-e // per_worker)
    nch = -(-nch // 2) * 2
    ep = nch * per_worker
    pad = ep - e

    src_p = jnp.concatenate([src, jnp.zeros((pad,), jnp.int32)])
    dst_p = jnp.concatenate([dst, jnp.ones((pad,), jnp.int32)])
    srcr = src_p.reshape(ep // _CHUNK, _CHUNK)
    dstr = dst_p.reshape(ep // _CHUNK, _CHUNK)

    xb = x1.astype(jnp.bfloat16)
    xu = lax.bitcast_convert_type(xb, jnp.uint16).astype(jnp.uint32)
    xu = jnp.concatenate([xu, jnp.zeros((2 * _XPW - _N,), jnp.uint32)])
    xp = xu[0::2] | (xu[1::2] << 16)
    xp = lax.bitcast_convert_type(xp, jnp.int32)
    zeros = jnp.zeros((_ACC_P,), jnp.float32)

    q = _make_sc_kernel(nch)(xp, srcr, dstr, zeros)

    xz = jnp.concatenate([x1[::3], jnp.zeros((_ACC_P - _K,), jnp.float32)])

    params = jnp.stack([
        W_l[0, 0], W_r[0, 0],
        W1[0, 0], b1[0], W1[1, 0], b1[1],
        W2[0, 0], W2[0, 1], b2[0],
    ])
    out2d = pl.pallas_call(
        _combine_body,
        out_shape=jax.ShapeDtypeStruct((_ACC_P // 128, 128), jnp.float32),
        in_specs=[
            pl.BlockSpec(memory_space=pltpu.SMEM),
            pl.BlockSpec(memory_space=pltpu.VMEM),
            pl.BlockSpec(memory_space=pltpu.VMEM),
        ],
    )(params, q.reshape(_NW, _ACC_P // 128, 128),
      xz.reshape(_ACC_P // 128, 128))
    return out2d.reshape(-1)[:_K]


# speedup vs baseline: 1.0310x; 1.0039x over previous
import functools

import jax
import jax.numpy as jnp
from jax import lax
from jax.experimental import pallas as pl
from jax.experimental.pallas import tpu as pltpu
from jax.experimental.pallas import tpu_sc as plsc

_N = 99999
_K = _N // 3
_NC = 2
_NS = 16
_NW = _NC * _NS
_CHUNK = 11264
_NV = _CHUNK // 16
_XPW = 50304
_ACC_P = 33408
_INV3 = 0xAAAAAAAB
_LIM3 = 0x55555555


def _make_sc_kernel(nch):
    mesh = plsc.VectorSubcoreMesh(core_axis_name="c", subcore_axis_name="s",
                                  num_cores=_NC, num_subcores=_NS)

    @functools.partial(
        pl.kernel,
        out_type=jax.ShapeDtypeStruct((_NW * _ACC_P,), jnp.float32),
        mesh=mesh,
        compiler_params=pltpu.CompilerParams(
            needs_layout_passes=False, use_tc_tiling_on_sc=False),
        scratch_types=[
            pltpu.VMEM((_XPW,), jnp.int32),
            pltpu.VMEM((2, _CHUNK), jnp.int32),
            pltpu.VMEM((2, _CHUNK), jnp.int32),
            pltpu.VMEM((_ACC_P,), jnp.float32),
            pltpu.SemaphoreType.DMA,
            pltpu.SemaphoreType.DMA,
            pltpu.SemaphoreType.DMA,
            pltpu.SemaphoreType.DMA,
        ],
    )
    def sc_kernel(xp_hbm, srcr_hbm, dstr_hbm, zeros_hbm, q_hbm,
                  xp_v, src_v, dst_v, acc_v,
                  sem_s0, sem_s1, sem_d0, sem_d1):
        cid = lax.axis_index("c")
        sid = lax.axis_index("s")
        wid = sid * _NC + cid
        sem_s = (sem_s0, sem_s1)
        sem_d = (sem_d0, sem_d1)

        row0 = wid * nch
        inv3 = jnp.full((16,), _INV3, dtype=jnp.uint32)
        lim3 = jnp.full((16,), _LIM3, dtype=jnp.uint32)
        himask = jnp.full((16,), 0xFFFF0000, dtype=jnp.uint32)
        one = jnp.full((16,), 1, dtype=jnp.int32)

        def start(g, b):
            pltpu.async_copy(srcr_hbm.at[row0 + g], src_v.at[b], sem_s[b])
            pltpu.async_copy(dstr_hbm.at[row0 + g], dst_v.at[b], sem_d[b])

        def wait(g, b):
            pltpu.make_async_copy(srcr_hbm.at[row0 + g], src_v.at[b],
                                  sem_s[b]).wait()
            pltpu.make_async_copy(dstr_hbm.at[row0 + g], dst_v.at[b],
                                  sem_d[b]).wait()

        def process(b):
            @plsc.parallel_loop(0, _NV, 1, unroll=8)
            def _body(u):
                off = u * 16
                d = dst_v[b, pl.ds(off, 16)]
                s = src_v[b, pl.ds(off, 16)]
                t = plsc.bitcast(d, jnp.uint32) * inv3
                m = t <= lim3
                pair = plsc.load_gather(xp_v, [lax.shift_right_logical(s, 1)])
                pu = plsc.bitcast(pair, jnp.uint32)
                sel = jnp.bitwise_and(s, one)
                bits = jnp.where(sel == one, pu & himask,
                                 lax.shift_left(pu, jnp.uint32(16)))
                v = plsc.bitcast(bits, jnp.float32)
                q = plsc.bitcast(t, jnp.int32)
                plsc.addupdate_scatter(acc_v, [q], v, mask=m)

        start(0, 0)
        start(1, 1)
        pltpu.sync_copy(xp_hbm, xp_v)
        pltpu.sync_copy(zeros_hbm, acc_v)

        def outer(i, carry):
            for b in range(2):
                g = 2 * i + b
                wait(g, b)
                process(b)

                @pl.when(g + 2 < nch)
                def _():
                    start(g + 2, b)
            return carry

        lax.fori_loop(0, nch // 2, outer, 0)

        pltpu.sync_copy(acc_v, q_hbm.at[pl.ds(wid * _ACC_P, _ACC_P)])

    return sc_kernel


def _combine_body(p_ref, q_ref, z_ref, o_ref):
    agg = jnp.sum(q_ref[...], axis=0)
    h = p_ref[0] * agg + p_ref[1] * z_ref[...]
    a0 = jnp.maximum(p_ref[2] * h + p_ref[3], 0.0)
    a1 = jnp.maximum(p_ref[4] * h + p_ref[5], 0.0)
    o_ref[...] = p_ref[6] * a0 + p_ref[7] * a1 + p_ref[8]


def kernel(x, edge_index, W_l, W_r, W1, b1, W2, b2):
    x1 = x.reshape(-1)
    src = edge_index[0]
    dst = edge_index[1]
    e = src.shape[0]

    per_worker = _NW * _CHUNK
    nch = -(-e // per_worker)
    nch = -(-nch // 2) * 2
    ep = nch * per_worker
    pad = ep - e

    src_p = jnp.concatenate([src, jnp.zeros((pad,), jnp.int32)])
    dst_p = jnp.concatenate([dst, jnp.ones((pad,), jnp.int32)])
    srcr = src_p.reshape(ep // _CHUNK, _CHUNK)
    dstr = dst_p.reshape(ep // _CHUNK, _CHUNK)

    xb = x1.astype(jnp.bfloat16)
    xu = lax.bitcast_convert_type(xb, jnp.uint16).astype(jnp.uint32)
    xu = jnp.concatenate([xu, jnp.zeros((2 * _XPW - _N,), jnp.uint32)])
    xp = xu[0::2] | (xu[1::2] << 16)
    xp = lax.bitcast_convert_type(xp, jnp.int32)
    zeros = jnp.zeros((_ACC_P,), jnp.float32)

    q = _make_sc_kernel(nch)(xp, srcr, dstr, zeros)

    xz = jnp.concatenate([x1[::3], jnp.zeros((_ACC_P - _K,), jnp.float32)])

    params = jnp.stack([
        W_l[0, 0], W_r[0, 0],
        W1[0, 0], b1[0], W1[1, 0], b1[1],
        W2[0, 0], W2[0, 1], b2[0],
    ])
    out2d = pl.pallas_call(
        _combine_body,
        out_shape=jax.ShapeDtypeStruct((_ACC_P // 128, 128), jnp.float32),
        in_specs=[
            pl.BlockSpec(memory_space=pltpu.SMEM),
            pl.BlockSpec(memory_space=pltpu.VMEM),
            pl.BlockSpec(memory_space=pltpu.VMEM),
        ],
    )(params, q.reshape(_NW, _ACC_P // 128, 128),
      xz.reshape(_ACC_P // 128, 128))
    return out2d.reshape(-1)[:_K]
